# Initial kernel scaffold; baseline (speedup 1.0000x reference)
import functools

import jax
import jax.numpy as jnp
from jax import lax
from jax.experimental import pallas as pl
from jax.experimental.pallas import tpu as pltpu
from jax.experimental.pallas import tpu_sc as plsc

N_NODES = 10000
N_EDGES = 320000
D_FEAT = 128
N_HIDDEN = 128
N_CLASSES = 64

NC = 2
NS = 16
NW = NC * NS
LANES = 16
CHUNK = 128
N_CHUNKS = N_EDGES // CHUNK
FULL_ITERS = N_CHUNKS // NW
REM = N_CHUNKS - FULL_ITERS * NW
ROWS_PER_TILE = N_NODES // NS
ZROWS = 125


def _sc_round_body(ft_hbm, src_hbm, dst_hbm, nrm_hbm, out_hbm,
                   sidx_v, didx_v, nrm_v, rows_v, acc, sem):
    cid = lax.axis_index("c")
    sid = lax.axis_index("s")
    wid = sid * NC + cid

    for r in range(ZROWS):
        for j in range(8):
            rows_v[r, pl.ds(j * LANES, LANES)] = jnp.zeros((LANES,), jnp.float32)
    tile_base = sid * ROWS_PER_TILE
    for z in range(5):
        pltpu.sync_copy(rows_v.at[pl.ds(0, ZROWS)],
                        acc.at[pl.ds(tile_base + z * ZROWS, ZROWS)])
    plsc.subcore_barrier()

    n_my = jnp.where(wid < REM, FULL_ITERS + 1, FULL_ITERS)

    @pl.loop(0, n_my)
    def _(i):
        g = i * NW + wid
        base = g * CHUNK
        pltpu.sync_copy(src_hbm.at[pl.ds(base, CHUNK)], sidx_v)
        pltpu.sync_copy(dst_hbm.at[pl.ds(base, CHUNK)], didx_v)
        pltpu.sync_copy(nrm_hbm.at[pl.ds(base, CHUNK)], nrm_v)
        pltpu.async_copy(ft_hbm.at[sidx_v], rows_v, sem).wait()
        for e in range(CHUNK):
            s = nrm_v[e]
            for j in range(8):
                slc = pl.ds(j * LANES, LANES)
                rows_v[e, slc] = rows_v[e, slc] * s
        pltpu.sync_copy(rows_v, acc.at[didx_v], add=True)

    plsc.subcore_barrier()

    pltpu.sync_copy(acc.at[pl.ds(tile_base, ROWS_PER_TILE)],
                    out_hbm.at[cid, pl.ds(tile_base, ROWS_PER_TILE)])


def _sc_round(ft, src, dst, nrm):
    mesh = plsc.VectorSubcoreMesh(core_axis_name="c", subcore_axis_name="s")
    kern = pl.kernel(
        _sc_round_body,
        out_type=jax.ShapeDtypeStruct((NC, N_NODES, D_FEAT), jnp.float32),
        mesh=mesh,
        scratch_types=[
            pltpu.VMEM((CHUNK,), jnp.int32),
            pltpu.VMEM((CHUNK,), jnp.int32),
            pltpu.VMEM((CHUNK,), jnp.float32),
            pltpu.VMEM((CHUNK, D_FEAT), jnp.float32),
            pltpu.VMEM_SHARED((N_NODES, D_FEAT), jnp.float32),
            pltpu.SemaphoreType.DMA,
        ],
    )
    return kern(ft, src, dst, nrm)


def _merge_body(p_ref, o_ref):
    o_ref[...] = p_ref[0] + p_ref[1]


def _merge(parts):
    return pl.pallas_call(
        _merge_body,
        out_shape=jax.ShapeDtypeStruct((N_NODES, D_FEAT), jnp.float32),
    )(parts)


def _mlp_body(p_ref, w1_ref, b1_ref, g_ref, be_ref, w2_ref, b2_ref, o_ref):
    ft = p_ref[0] + p_ref[1]
    h = lax.dot_general(ft, w1_ref[...], (((1,), (1,)), ((), ())),
                        precision=lax.Precision.HIGHEST,
                        preferred_element_type=jnp.float32)
    h = h + b1_ref[...][None, :]
    mean = jnp.mean(h, axis=0)
    var = jnp.mean(jnp.square(h), axis=0) - jnp.square(mean)
    h = (h - mean[None, :]) * (g_ref[...] / jnp.sqrt(var + 1e-5))[None, :]
    h = h + be_ref[...][None, :]
    h = jnp.maximum(h, 0.0)
    o = lax.dot_general(h, w2_ref[...], (((1,), (1,)), ((), ())),
                        precision=lax.Precision.HIGHEST,
                        preferred_element_type=jnp.float32)
    o_ref[...] = o + b2_ref[...][None, :]


def _mlp(parts, W1, b1, gamma, beta, W2, b2):
    return pl.pallas_call(
        _mlp_body,
        out_shape=jax.ShapeDtypeStruct((N_NODES, N_CLASSES), jnp.float32),
    )(parts, W1, b1, gamma, beta, W2, b2)


def kernel(feat, edge_index, gcn_norm, W1, b1, gamma, beta, W2, b2):
    src = edge_index[0].astype(jnp.int32)
    dst = edge_index[1].astype(jnp.int32)
    parts = _sc_round(feat, src, dst, gcn_norm)
    for _ in range(2):
        ft = _merge(parts)
        parts = _sc_round(ft, src, dst, gcn_norm)
    return _mlp(parts, W1, b1, gamma, beta, W2, b2)


# speedup vs baseline: 4.6124x; 4.6124x over previous
import functools

import jax
import jax.numpy as jnp
from jax import lax
from jax.experimental import pallas as pl
from jax.experimental.pallas import tpu as pltpu
from jax.experimental.pallas import tpu_sc as plsc

N_NODES = 10000
N_EDGES = 320000
D_FEAT = 128
N_HIDDEN = 128
N_CLASSES = 64

NC = 2
NS = 16
NW = NC * NS
LANES = 16
CHUNK = 128
N_CHUNKS = N_EDGES // CHUNK
FULL_ITERS = N_CHUNKS // NW
REM = N_CHUNKS - FULL_ITERS * NW
N_PAD = 10240
ROWS_PER_TILE = N_PAD // NS


def _sc_round_body(ft_hbm, src_hbm, dst_hbm, nrm_hbm, out_hbm,
                   sidx_v, didx_v, nrm_v, rows_v, acc, sem):
    cid = lax.axis_index("c")
    sid = lax.axis_index("s")
    wid = sid * NC + cid

    for r in range(CHUNK):
        for j in range(8):
            rows_v[r, pl.ds(j * LANES, LANES)] = jnp.zeros((LANES,), jnp.float32)
    tile_base = pl.multiple_of(sid * ROWS_PER_TILE, ROWS_PER_TILE)
    for z in range(ROWS_PER_TILE // CHUNK):
        pltpu.sync_copy(rows_v,
                        acc.at[pl.ds(pl.multiple_of(tile_base + z * CHUNK, CHUNK),
                                     CHUNK)])
    plsc.subcore_barrier()

    n_my = jnp.where(wid < REM, FULL_ITERS + 1, FULL_ITERS)

    @pl.loop(0, n_my)
    def _(i):
        g = i * NW + wid
        base = g * CHUNK
        pltpu.sync_copy(src_hbm.at[pl.ds(base, CHUNK)], sidx_v)
        pltpu.sync_copy(dst_hbm.at[pl.ds(base, CHUNK)], didx_v)
        pltpu.sync_copy(nrm_hbm.at[pl.ds(base, CHUNK)], nrm_v)
        pltpu.async_copy(ft_hbm.at[sidx_v], rows_v, sem).wait()
        for t in range(CHUNK // LANES):
            nv = nrm_v[pl.ds(t * LANES, LANES)]
            for el in range(LANES):
                e = t * LANES + el
                s = nv[el]
                for j in range(8):
                    slc = pl.ds(j * LANES, LANES)
                    rows_v[e, slc] = rows_v[e, slc] * s
        pltpu.sync_copy(rows_v, acc.at[didx_v], add=True)

    plsc.subcore_barrier()

    pltpu.sync_copy(acc.at[pl.ds(tile_base, ROWS_PER_TILE)],
                    out_hbm.at[cid, pl.ds(tile_base, ROWS_PER_TILE)])


def _sc_round(ft, src, dst, nrm):
    mesh = plsc.VectorSubcoreMesh(core_axis_name="c", subcore_axis_name="s")
    kern = pl.kernel(
        _sc_round_body,
        out_type=jax.ShapeDtypeStruct((NC, N_PAD, D_FEAT), jnp.float32),
        mesh=mesh,
        scratch_types=[
            pltpu.VMEM((CHUNK,), jnp.int32),
            pltpu.VMEM((CHUNK,), jnp.int32),
            pltpu.VMEM((CHUNK,), jnp.float32),
            pltpu.VMEM((CHUNK, D_FEAT), jnp.float32),
            pltpu.VMEM_SHARED((N_PAD, D_FEAT), jnp.float32),
            pltpu.SemaphoreType.DMA,
        ],
    )
    return kern(ft, src, dst, nrm)


def _merge_body(p_ref, o_ref):
    o_ref[...] = p_ref[0, :N_NODES] + p_ref[1, :N_NODES]


def _merge(parts):
    return pl.pallas_call(
        _merge_body,
        out_shape=jax.ShapeDtypeStruct((N_NODES, D_FEAT), jnp.float32),
    )(parts)


def _mlp_body(p_ref, w1_ref, b1_ref, g_ref, be_ref, w2_ref, b2_ref, o_ref):
    ft = p_ref[0, :N_NODES] + p_ref[1, :N_NODES]
    h = lax.dot_general(ft, w1_ref[...], (((1,), (1,)), ((), ())),
                        precision=lax.Precision.HIGHEST,
                        preferred_element_type=jnp.float32)
    h = h + b1_ref[...][None, :]
    mean = jnp.mean(h, axis=0)
    var = jnp.mean(jnp.square(h), axis=0) - jnp.square(mean)
    h = (h - mean[None, :]) * (g_ref[...] / jnp.sqrt(var + 1e-5))[None, :]
    h = h + be_ref[...][None, :]
    h = jnp.maximum(h, 0.0)
    o = lax.dot_general(h, w2_ref[...], (((1,), (1,)), ((), ())),
                        precision=lax.Precision.HIGHEST,
                        preferred_element_type=jnp.float32)
    o_ref[...] = o + b2_ref[...][None, :]


def _mlp(parts, W1, b1, gamma, beta, W2, b2):
    return pl.pallas_call(
        _mlp_body,
        out_shape=jax.ShapeDtypeStruct((N_NODES, N_CLASSES), jnp.float32),
    )(parts, W1, b1, gamma, beta, W2, b2)


def kernel(feat, edge_index, gcn_norm, W1, b1, gamma, beta, W2, b2):
    src = edge_index[0].astype(jnp.int32)
    dst = edge_index[1].astype(jnp.int32)
    parts = _sc_round(feat, src, dst, gcn_norm)
    for _ in range(2):
        ft = _merge(parts)
        parts = _sc_round(ft, src, dst, gcn_norm)
    return _mlp(parts, W1, b1, gamma, beta, W2, b2)
